# baseline (speedup 1.0000x reference)
import jax
import jax.numpy as jnp
import numpy as np
from jax.experimental import pallas as pl
from jax.experimental.pallas import tpu as pltpu

_B = 16384
_R = 128
_C = 128
_LOG2B = 14

_TABLES = None


def _threefry2x32(k1, k2, x0, x1):
    u32 = np.uint32

    def rotl(v, d):
        return ((v << u32(d)) | (v >> u32(32 - d))).astype(np.uint32)

    ks0, ks1 = u32(k1), u32(k2)
    ks2 = u32(ks0 ^ ks1 ^ u32(0x1BD11BDA))
    rot_a = (13, 15, 26, 6)
    rot_b = (17, 29, 16, 24)

    def four_rounds(a, b, rots):
        for r in rots:
            a = (a + b).astype(np.uint32)
            b = rotl(b, r)
            b = (a ^ b).astype(np.uint32)
        return a, b

    a = (x0 + ks0).astype(np.uint32)
    b = (x1 + ks1).astype(np.uint32)
    a, b = four_rounds(a, b, rot_a)
    a, b = (a + ks1).astype(np.uint32), (b + ks2 + u32(1)).astype(np.uint32)
    a, b = four_rounds(a, b, rot_b)
    a, b = (a + ks2).astype(np.uint32), (b + ks0 + u32(2)).astype(np.uint32)
    a, b = four_rounds(a, b, rot_a)
    a, b = (a + ks0).astype(np.uint32), (b + ks1 + u32(3)).astype(np.uint32)
    a, b = four_rounds(a, b, rot_b)
    a, b = (a + ks1).astype(np.uint32), (b + ks2 + u32(4)).astype(np.uint32)
    a, b = four_rounds(a, b, rot_a)
    a, b = (a + ks2).astype(np.uint32), (b + ks0 + u32(5)).astype(np.uint32)
    return a, b


def _tables():
    global _TABLES
    if _TABLES is None:
        n = _B + 1
        hi = np.zeros(n, dtype=np.uint32)
        lo = np.arange(n, dtype=np.uint32)
        a, b = _threefry2x32(0, 1234, hi, lo)
        bits = a ^ b
        fbits = (bits >> np.uint32(9)) | np.uint32(0x3F800000)
        u = fbits.view(np.float32) - np.float32(1.0)
        rd = np.sort(-(-np.log1p(-u)))
        r = np.arange(_R)[:, None]
        c = np.arange(_C)[None, :]
        crev = np.zeros_like(c)
        for j in range(7):
            crev = crev | (((c >> (6 - j)) & 1) << j)
        k = (r >> 3) | ((r & 7) << 4) | (crev << 7)
        _TABLES = (
            np.ascontiguousarray(rd[:_B][k]),
            np.ascontiguousarray(rd[1 : _B + 1][k]),
        )
    return _TABLES


def _cx_blocks(v, size, axis):
    n = v.shape[axis]
    parts = []
    for b in range(0, n // size, 2):
        if axis == 0:
            a = v[b * size : (b + 1) * size]
            c = v[(b + 1) * size : (b + 2) * size]
        else:
            a = v[:, b * size : (b + 1) * size]
            c = v[:, (b + 1) * size : (b + 2) * size]
        parts.append(jnp.minimum(a, c))
        parts.append(jnp.maximum(a, c))
    return jnp.concatenate(parts, axis=axis)


def _coto_body(out_ref, tgt_ref, t0_ref, t1_ref, loss_ref):
    x = out_ref[:]
    t = 2.0 * tgt_ref[:].astype(jnp.float32) - 1.0
    s = x * t

    row = jax.lax.broadcasted_iota(jnp.int32, (_R, _C), 0)
    col = jax.lax.broadcasted_iota(jnp.int32, (_R, _C), 1)
    crev = jnp.zeros_like(col)
    for j in range(7):
        crev = crev | (((col >> (6 - j)) & 1) << j)
    idx = (row >> 3) | ((row & 7) << 4) | (crev << 7)

    r0 = jnp.sum((s <= 0.0).astype(jnp.float32)).astype(jnp.int32)

    gray = idx ^ (idx >> 1)

    def flip_signs(w, bits):
        wi = jax.lax.bitcast_convert_type(w, jnp.int32)
        return jax.lax.bitcast_convert_type(wi ^ bits, jnp.float32)

    v = flip_signs(s, (idx & 2) << 30)
    for p in range(_LOG2B):
        if p > 0:
            v = flip_signs(v, (gray & (1 << p)) << (31 - p))
        for q in range(p, -1, -1):
            d = 1 << q
            if d < 16:
                v = _cx_blocks(v, 8 * d, axis=0)
            elif d < 128:
                sh = d >> 4
                lower = (row & sh) == 0
                up = jnp.roll(v, -sh, axis=0)
                dn = jnp.roll(v, sh, axis=0)
                v = jnp.where(lower, jnp.minimum(v, up), jnp.maximum(v, dn))
            else:
                dc = 8192 // d
                if dc >= 16:
                    v = _cx_blocks(v, dc, axis=1)
                else:
                    lower = (col & dc) == 0
                    up = jnp.roll(v, -dc, axis=1)
                    dn = jnp.roll(v, dc, axis=1)
                    v = jnp.where(lower, jnp.minimum(v, up), jnp.maximum(v, dn))

    t0 = t0_ref[:]
    t1 = t1_ref[:]

    rd_r0 = jnp.sum(jnp.where(idx == r0, t0, 0.0)) + jnp.where(
        r0 == _B, 1.0, 0.0
    ) * jnp.sum(jnp.where(idx == _B - 1, t1, 0.0))
    inv_d = -1.0 / rd_r0

    c = jnp.where(idx < r0, t0, t1)
    g = c * inv_d
    g = jnp.where(v > 1.0, jnp.maximum(-jnp.exp(-v), g), g)
    loss_ref[0, 0] = jnp.sum(g * v) * (1.0 / _B)


def kernel(output, target):
    t0, t1 = _tables()
    out2 = output.reshape(_R, _C)
    tgt2 = target.astype(jnp.int32).reshape(_R, _C)
    loss = pl.pallas_call(
        _coto_body,
        out_shape=jax.ShapeDtypeStruct((1, 1), jnp.float32),
        in_specs=[
            pl.BlockSpec(memory_space=pltpu.VMEM),
            pl.BlockSpec(memory_space=pltpu.VMEM),
            pl.BlockSpec(memory_space=pltpu.VMEM),
            pl.BlockSpec(memory_space=pltpu.VMEM),
        ],
        out_specs=pl.BlockSpec(memory_space=pltpu.SMEM),
    )(out2, tgt2, jnp.asarray(t0), jnp.asarray(t1))
    return loss[0, 0]


# speedup vs baseline: 1.1946x; 1.1946x over previous
import jax
import jax.numpy as jnp
import numpy as np
from jax.experimental import pallas as pl
from jax.experimental.pallas import tpu as pltpu

_B = 16384
_R = 128
_C = 128
_LOG2B = 14

_TABLES = None


def _threefry2x32(k1, k2, x0, x1):
    u32 = np.uint32

    def rotl(v, d):
        return ((v << u32(d)) | (v >> u32(32 - d))).astype(np.uint32)

    ks0, ks1 = u32(k1), u32(k2)
    ks2 = u32(ks0 ^ ks1 ^ u32(0x1BD11BDA))
    rot_a = (13, 15, 26, 6)
    rot_b = (17, 29, 16, 24)

    def four_rounds(a, b, rots):
        for r in rots:
            a = (a + b).astype(np.uint32)
            b = rotl(b, r)
            b = (a ^ b).astype(np.uint32)
        return a, b

    a = (x0 + ks0).astype(np.uint32)
    b = (x1 + ks1).astype(np.uint32)
    a, b = four_rounds(a, b, rot_a)
    a, b = (a + ks1).astype(np.uint32), (b + ks2 + u32(1)).astype(np.uint32)
    a, b = four_rounds(a, b, rot_b)
    a, b = (a + ks2).astype(np.uint32), (b + ks0 + u32(2)).astype(np.uint32)
    a, b = four_rounds(a, b, rot_a)
    a, b = (a + ks0).astype(np.uint32), (b + ks1 + u32(3)).astype(np.uint32)
    a, b = four_rounds(a, b, rot_b)
    a, b = (a + ks1).astype(np.uint32), (b + ks2 + u32(4)).astype(np.uint32)
    a, b = four_rounds(a, b, rot_a)
    a, b = (a + ks2).astype(np.uint32), (b + ks0 + u32(5)).astype(np.uint32)
    return a, b


def _tables():
    global _TABLES
    if _TABLES is None:
        n = _B + 1
        hi = np.zeros(n, dtype=np.uint32)
        lo = np.arange(n, dtype=np.uint32)
        a, b = _threefry2x32(0, 1234, hi, lo)
        bits = a ^ b
        fbits = (bits >> np.uint32(9)) | np.uint32(0x3F800000)
        u = fbits.view(np.float32) - np.float32(1.0)
        rd = np.sort(-(-np.log1p(-u)))
        r = np.arange(_R)[:, None]
        c = np.arange(_C)[None, :]
        k = (r >> 3) | ((r & 7) << 4) | (c << 7)
        _TABLES = (
            np.ascontiguousarray(rd[:_B][k]),
            np.ascontiguousarray(rd[1 : _B + 1][k]),
        )
    return _TABLES


def _cx_blocks(v, size, axis):
    n = v.shape[axis]
    parts = []
    for b in range(0, n // size, 2):
        if axis == 0:
            a = v[b * size : (b + 1) * size]
            c = v[(b + 1) * size : (b + 2) * size]
        else:
            a = v[:, b * size : (b + 1) * size]
            c = v[:, (b + 1) * size : (b + 2) * size]
        parts.append(jnp.minimum(a, c))
        parts.append(jnp.maximum(a, c))
    return jnp.concatenate(parts, axis=axis)


def _coto_body(out_ref, tgt_ref, t0_ref, t1_ref, loss_ref):
    x = out_ref[:]
    t = 2.0 * tgt_ref[:].astype(jnp.float32) - 1.0
    s = x * t

    row = jax.lax.broadcasted_iota(jnp.int32, (_R, _C), 0)
    col = jax.lax.broadcasted_iota(jnp.int32, (_R, _C), 1)
    idx = (row >> 3) | ((row & 7) << 4) | (col << 7)

    r0 = jnp.sum((s <= 0.0).astype(jnp.float32)).astype(jnp.int32)

    gray = idx ^ (idx >> 1)

    def flip_signs(w, bits):
        wi = jax.lax.bitcast_convert_type(w, jnp.int32)
        return jax.lax.bitcast_convert_type(wi ^ bits, jnp.float32)

    v = flip_signs(s, (idx & 2) << 30)
    for p in range(_LOG2B):
        if p > 0:
            v = flip_signs(v, (gray & (1 << p)) << (31 - p))
        for q in range(p, -1, -1):
            d = 1 << q
            if d < 16:
                v = _cx_blocks(v, 8 * d, axis=0)
            elif d < 128:
                sh = d >> 4
                lower = (row & sh) == 0
                up = jnp.roll(v, -sh, axis=0)
                dn = jnp.roll(v, sh, axis=0)
                v = jnp.where(lower, jnp.minimum(v, up), jnp.maximum(v, dn))
            else:
                dc = d >> 7
                lower = (col & dc) == 0
                up = jnp.roll(v, -dc, axis=1)
                dn = jnp.roll(v, dc, axis=1)
                v = jnp.where(lower, jnp.minimum(v, up), jnp.maximum(v, dn))

    t0 = t0_ref[:]
    t1 = t1_ref[:]

    rd_r0 = jnp.sum(jnp.where(idx == r0, t0, 0.0)) + jnp.where(
        r0 == _B, 1.0, 0.0
    ) * jnp.sum(jnp.where(idx == _B - 1, t1, 0.0))
    inv_d = -1.0 / rd_r0

    c = jnp.where(idx < r0, t0, t1)
    g = c * inv_d
    g = jnp.where(v > 1.0, jnp.maximum(-jnp.exp(-v), g), g)
    loss_ref[0, 0] = jnp.sum(g * v) * (1.0 / _B)


def kernel(output, target):
    t0, t1 = _tables()
    out2 = output.reshape(_R, _C)
    tgt2 = target.astype(jnp.int32).reshape(_R, _C)
    loss = pl.pallas_call(
        _coto_body,
        out_shape=jax.ShapeDtypeStruct((1, 1), jnp.float32),
        in_specs=[
            pl.BlockSpec(memory_space=pltpu.VMEM),
            pl.BlockSpec(memory_space=pltpu.VMEM),
            pl.BlockSpec(memory_space=pltpu.VMEM),
            pl.BlockSpec(memory_space=pltpu.VMEM),
        ],
        out_specs=pl.BlockSpec(memory_space=pltpu.SMEM),
    )(out2, tgt2, jnp.asarray(t0), jnp.asarray(t1))
    return loss[0, 0]


# speedup vs baseline: 1.1955x; 1.0008x over previous
import jax
import jax.numpy as jnp
import numpy as np
from jax.experimental import pallas as pl
from jax.experimental.pallas import tpu as pltpu

_B = 16384
_R = 128
_C = 128
_LOG2B = 14

_TABLES = None


def _threefry2x32(k1, k2, x0, x1):
    u32 = np.uint32

    def rotl(v, d):
        return ((v << u32(d)) | (v >> u32(32 - d))).astype(np.uint32)

    ks0, ks1 = u32(k1), u32(k2)
    ks2 = u32(ks0 ^ ks1 ^ u32(0x1BD11BDA))
    rot_a = (13, 15, 26, 6)
    rot_b = (17, 29, 16, 24)

    def four_rounds(a, b, rots):
        for r in rots:
            a = (a + b).astype(np.uint32)
            b = rotl(b, r)
            b = (a ^ b).astype(np.uint32)
        return a, b

    a = (x0 + ks0).astype(np.uint32)
    b = (x1 + ks1).astype(np.uint32)
    a, b = four_rounds(a, b, rot_a)
    a, b = (a + ks1).astype(np.uint32), (b + ks2 + u32(1)).astype(np.uint32)
    a, b = four_rounds(a, b, rot_b)
    a, b = (a + ks2).astype(np.uint32), (b + ks0 + u32(2)).astype(np.uint32)
    a, b = four_rounds(a, b, rot_a)
    a, b = (a + ks0).astype(np.uint32), (b + ks1 + u32(3)).astype(np.uint32)
    a, b = four_rounds(a, b, rot_b)
    a, b = (a + ks1).astype(np.uint32), (b + ks2 + u32(4)).astype(np.uint32)
    a, b = four_rounds(a, b, rot_a)
    a, b = (a + ks2).astype(np.uint32), (b + ks0 + u32(5)).astype(np.uint32)
    return a, b


def _tables():
    global _TABLES
    if _TABLES is None:
        n = _B + 1
        hi = np.zeros(n, dtype=np.uint32)
        lo = np.arange(n, dtype=np.uint32)
        a, b = _threefry2x32(0, 1234, hi, lo)
        bits = a ^ b
        fbits = (bits >> np.uint32(9)) | np.uint32(0x3F800000)
        u = fbits.view(np.float32) - np.float32(1.0)
        rd = np.sort(-(-np.log1p(-u)))
        r = np.arange(_R)[:, None]
        c = np.arange(_C)[None, :]
        k = (r >> 3) | ((r & 7) << 4) | (c << 7)
        _TABLES = (
            np.ascontiguousarray(rd[:_B][k]),
            np.ascontiguousarray(rd[1 : _B + 1][k]),
        )
    return _TABLES


def _cx_blocks(v, size, axis):
    n = v.shape[axis]
    parts = []
    for b in range(0, n // size, 2):
        if axis == 0:
            a = v[b * size : (b + 1) * size]
            c = v[(b + 1) * size : (b + 2) * size]
        else:
            a = v[:, b * size : (b + 1) * size]
            c = v[:, (b + 1) * size : (b + 2) * size]
        parts.append(jnp.minimum(a, c))
        parts.append(jnp.maximum(a, c))
    return jnp.concatenate(parts, axis=axis)


def _coto_body(out_ref, tgt_ref, t0_ref, t1_ref, loss_ref):
    x = out_ref[:]
    t = 2.0 * tgt_ref[:].astype(jnp.float32) - 1.0
    s = x * t

    row = jax.lax.broadcasted_iota(jnp.int32, (_R, _C), 0)
    col = jax.lax.broadcasted_iota(jnp.int32, (_R, _C), 1)

    def idx_bit(k):
        if k <= 3:
            return (row >> (k + 3)) & 1
        if k <= 6:
            return (row >> (k - 4)) & 1
        return (col >> (k - 7)) & 1

    r0 = jnp.sum((s <= 0.0).astype(jnp.float32)).astype(jnp.int32)

    def flip_signs(w, bits):
        wi = jax.lax.bitcast_convert_type(w, jnp.int32)
        return jax.lax.bitcast_convert_type(wi ^ bits, jnp.float32)

    v = flip_signs(s, idx_bit(1) << 31)
    for p in range(_LOG2B):
        if p > 0:
            v = flip_signs(v, (idx_bit(p) ^ idx_bit(p + 1)) << 31)
        for q in range(p, -1, -1):
            d = 1 << q
            if d < 16:
                v = _cx_blocks(v, 8 * d, axis=0)
            elif d < 128:
                sh = d >> 4
                lower = (row & sh) == 0
                up = jnp.roll(v, -sh, axis=0)
                dn = jnp.roll(v, sh, axis=0)
                v = jnp.where(lower, jnp.minimum(v, up), jnp.maximum(v, dn))
            else:
                dc = d >> 7
                lower = (col & dc) == 0
                up = jnp.roll(v, -dc, axis=1)
                dn = jnp.roll(v, dc, axis=1)
                v = jnp.where(lower, jnp.minimum(v, up), jnp.maximum(v, dn))

    idx = (row >> 3) | ((row & 7) << 4) | (col << 7)
    t0 = t0_ref[:]
    t1 = t1_ref[:]

    rd_r0 = jnp.sum(jnp.where(idx == r0, t0, 0.0)) + jnp.where(
        r0 == _B, 1.0, 0.0
    ) * jnp.sum(jnp.where(idx == _B - 1, t1, 0.0))
    inv_d = -1.0 / rd_r0

    c = jnp.where(idx < r0, t0, t1)
    g = c * inv_d
    g = jnp.where(v > 1.0, jnp.maximum(-jnp.exp(-v), g), g)
    loss_ref[0, 0] = jnp.sum(g * v) * (1.0 / _B)


def kernel(output, target):
    t0, t1 = _tables()
    out2 = output.reshape(_R, _C)
    tgt2 = target.astype(jnp.int32).reshape(_R, _C)
    loss = pl.pallas_call(
        _coto_body,
        out_shape=jax.ShapeDtypeStruct((1, 1), jnp.float32),
        in_specs=[
            pl.BlockSpec(memory_space=pltpu.VMEM),
            pl.BlockSpec(memory_space=pltpu.VMEM),
            pl.BlockSpec(memory_space=pltpu.VMEM),
            pl.BlockSpec(memory_space=pltpu.VMEM),
        ],
        out_specs=pl.BlockSpec(memory_space=pltpu.SMEM),
    )(out2, tgt2, jnp.asarray(t0), jnp.asarray(t1))
    return loss[0, 0]


# speedup vs baseline: 1.2359x; 1.0338x over previous
import jax
import jax.numpy as jnp
import numpy as np
from jax.experimental import pallas as pl
from jax.experimental.pallas import tpu as pltpu

_B = 16384
_R = 128
_C = 128
_LOG2B = 14

_TABLES = None


def _threefry2x32(k1, k2, x0, x1):
    u32 = np.uint32

    def rotl(v, d):
        return ((v << u32(d)) | (v >> u32(32 - d))).astype(np.uint32)

    ks0, ks1 = u32(k1), u32(k2)
    ks2 = u32(ks0 ^ ks1 ^ u32(0x1BD11BDA))
    rot_a = (13, 15, 26, 6)
    rot_b = (17, 29, 16, 24)

    def four_rounds(a, b, rots):
        for r in rots:
            a = (a + b).astype(np.uint32)
            b = rotl(b, r)
            b = (a ^ b).astype(np.uint32)
        return a, b

    a = (x0 + ks0).astype(np.uint32)
    b = (x1 + ks1).astype(np.uint32)
    a, b = four_rounds(a, b, rot_a)
    a, b = (a + ks1).astype(np.uint32), (b + ks2 + u32(1)).astype(np.uint32)
    a, b = four_rounds(a, b, rot_b)
    a, b = (a + ks2).astype(np.uint32), (b + ks0 + u32(2)).astype(np.uint32)
    a, b = four_rounds(a, b, rot_a)
    a, b = (a + ks0).astype(np.uint32), (b + ks1 + u32(3)).astype(np.uint32)
    a, b = four_rounds(a, b, rot_b)
    a, b = (a + ks1).astype(np.uint32), (b + ks2 + u32(4)).astype(np.uint32)
    a, b = four_rounds(a, b, rot_a)
    a, b = (a + ks2).astype(np.uint32), (b + ks0 + u32(5)).astype(np.uint32)
    return a, b


def _tables():
    global _TABLES
    if _TABLES is None:
        n = _B + 1
        hi = np.zeros(n, dtype=np.uint32)
        lo = np.arange(n, dtype=np.uint32)
        a, b = _threefry2x32(0, 1234, hi, lo)
        bits = a ^ b
        fbits = (bits >> np.uint32(9)) | np.uint32(0x3F800000)
        u = fbits.view(np.float32) - np.float32(1.0)
        rd = np.sort(-(-np.log1p(-u)))
        r = np.arange(_R)[:, None]
        c = np.arange(_C)[None, :]
        k = (r >> 3) | ((r & 7) << 4) | (c << 7)
        _TABLES = (
            np.ascontiguousarray(rd[:_B][k]),
            np.ascontiguousarray(rd[1 : _B + 1][k]),
        )
    return _TABLES


def _cx_blocks(v, size, axis):
    n = v.shape[axis]
    parts = []
    for b in range(0, n // size, 2):
        if axis == 0:
            a = v[b * size : (b + 1) * size]
            c = v[(b + 1) * size : (b + 2) * size]
        else:
            a = v[:, b * size : (b + 1) * size]
            c = v[:, (b + 1) * size : (b + 2) * size]
        parts.append(jnp.minimum(a, c))
        parts.append(jnp.maximum(a, c))
    return jnp.concatenate(parts, axis=axis)


def _coto_body(out_ref, tgt_ref, t0_ref, t1_ref, loss_ref):
    x = out_ref[:]
    t = 2.0 * tgt_ref[:].astype(jnp.float32) - 1.0
    s = x * t

    row = jax.lax.broadcasted_iota(jnp.int32, (_R, _C), 0)
    col = jax.lax.broadcasted_iota(jnp.int32, (_R, _C), 1)

    def idx_bit(k):
        if k <= 3:
            return (row >> (k + 3)) & 1
        if k <= 6:
            return (row >> (k - 4)) & 1
        return (col >> (k - 7)) & 1

    r0 = jnp.sum((s <= 0.0).astype(jnp.float32)).astype(jnp.int32)

    def flip_signs(w, bits):
        wi = jax.lax.bitcast_convert_type(w, jnp.int32)
        return jax.lax.bitcast_convert_type(wi ^ bits, jnp.float32)

    v = flip_signs(s, idx_bit(1) << 31)
    for p in range(_LOG2B):
        if p > 0:
            v = flip_signs(v, (idx_bit(p) ^ idx_bit(p + 1)) << 31)
        for q in range(p, -1, -1):
            d = 1 << q
            if d < 16:
                v = _cx_blocks(v, 8 * d, axis=0)
            elif d < 128:
                sh = d >> 4
                lower = (row & sh) == 0
                parts = []
                for b in range(_R // 8):
                    blk = v[8 * b : 8 * b + 8]
                    lw = lower[8 * b : 8 * b + 8]
                    up = jnp.roll(blk, -sh, axis=0)
                    dn = jnp.roll(blk, sh, axis=0)
                    parts.append(
                        jnp.where(lw, jnp.minimum(blk, up), jnp.maximum(blk, dn))
                    )
                v = jnp.concatenate(parts, axis=0)
            else:
                dc = d >> 7
                lower = (col & dc) == 0
                up = jnp.roll(v, -dc, axis=1)
                dn = jnp.roll(v, dc, axis=1)
                v = jnp.where(lower, jnp.minimum(v, up), jnp.maximum(v, dn))

    idx = (row >> 3) | ((row & 7) << 4) | (col << 7)
    t0 = t0_ref[:]
    t1 = t1_ref[:]

    rd_r0 = jnp.sum(jnp.where(idx == r0, t0, 0.0)) + jnp.where(
        r0 == _B, 1.0, 0.0
    ) * jnp.sum(jnp.where(idx == _B - 1, t1, 0.0))
    inv_d = -1.0 / rd_r0

    c = jnp.where(idx < r0, t0, t1)
    g = c * inv_d
    g = jnp.where(v > 1.0, jnp.maximum(-jnp.exp(-v), g), g)
    loss_ref[0, 0] = jnp.sum(g * v) * (1.0 / _B)


def kernel(output, target):
    t0, t1 = _tables()
    out2 = output.reshape(_R, _C)
    tgt2 = target.astype(jnp.int32).reshape(_R, _C)
    loss = pl.pallas_call(
        _coto_body,
        out_shape=jax.ShapeDtypeStruct((1, 1), jnp.float32),
        in_specs=[
            pl.BlockSpec(memory_space=pltpu.VMEM),
            pl.BlockSpec(memory_space=pltpu.VMEM),
            pl.BlockSpec(memory_space=pltpu.VMEM),
            pl.BlockSpec(memory_space=pltpu.VMEM),
        ],
        out_specs=pl.BlockSpec(memory_space=pltpu.SMEM),
    )(out2, tgt2, jnp.asarray(t0), jnp.asarray(t1))
    return loss[0, 0]
